# Initial kernel scaffold; baseline (speedup 1.0000x reference)
import jax
import jax.numpy as jnp
from jax.experimental import pallas as pl


def kernel(scores, bbox_deltas, im_info, cfg_key):
    raise NotImplementedError("write your pallas kernel here")


# speedup vs baseline: 1.1417x; 1.1417x over previous
import functools

import jax
import jax.numpy as jnp
import numpy as np
from jax.experimental import pallas as pl
from jax.experimental.pallas import tpu as pltpu

_A = 9
_FEAT_STRIDE = 16
_PRE_N = 6000
_POST_N = 300
_THRESH = 0.7
_NPAD = 6144
_ROWS = 8
_COLS = _NPAD // _ROWS


def _np_generate_anchors():
    base_size = 16
    ratios = np.array([0.5, 1.0, 2.0])
    scales = np.array([8.0, 16.0, 32.0])

    def whctrs(a):
        w = a[2] - a[0] + 1.0
        h = a[3] - a[1] + 1.0
        return w, h, a[0] + 0.5 * (w - 1.0), a[1] + 0.5 * (h - 1.0)

    def mkanchors(ws, hs, xc, yc):
        ws = ws[:, None]
        hs = hs[:, None]
        return np.hstack((xc - 0.5 * (ws - 1.0), yc - 0.5 * (hs - 1.0),
                          xc + 0.5 * (ws - 1.0), yc + 0.5 * (hs - 1.0)))

    base = np.array([1.0, 1.0, base_size, base_size], dtype=np.float64) - 1.0
    w, h, xc, yc = whctrs(base)
    size = w * h
    ws = np.round(np.sqrt(size / ratios))
    hs = np.round(ws * ratios)
    ra = mkanchors(ws, hs, xc, yc)
    outs = []
    for i in range(ra.shape[0]):
        w, h, xc, yc = whctrs(ra[i])
        outs.append(mkanchors(w * scales, h * scales, xc, yc))
    return np.vstack(outs)


def _nms_body(boxes_ref, scores_ref, out_ref):
    x1 = boxes_ref[0, 0, :, :]
    y1 = boxes_ref[0, 1, :, :]
    x2 = boxes_ref[0, 2, :, :]
    y2 = boxes_ref[0, 3, :, :]
    areas = (x2 - x1 + 1.0) * (y2 - y1 + 1.0)
    s0 = scores_ref[0, :, :]

    ri = jax.lax.broadcasted_iota(jnp.int32, (_ROWS, _COLS), 0)
    ci = jax.lax.broadcasted_iota(jnp.int32, (_ROWS, _COLS), 1)
    fi = ri * _COLS + ci
    neg_inf = jnp.float32(-jnp.inf)

    def step(t, s):
        m = jnp.max(s)
        ok = jnp.isfinite(m)
        eq = s == m
        idx = jnp.min(jnp.where(eq, fi, jnp.int32(2**30)))
        sel = fi == idx
        zero = jnp.float32(0.0)
        bx1 = jnp.sum(jnp.where(sel, x1, zero))
        by1 = jnp.sum(jnp.where(sel, y1, zero))
        bx2 = jnp.sum(jnp.where(sel, x2, zero))
        by2 = jnp.sum(jnp.where(sel, y2, zero))
        barea = jnp.sum(jnp.where(sel, areas, zero))
        xx1 = jnp.maximum(x1, bx1)
        yy1 = jnp.maximum(y1, by1)
        xx2 = jnp.minimum(x2, bx2)
        yy2 = jnp.minimum(y2, by2)
        inter = (jnp.maximum(0.0, xx2 - xx1 + 1.0)
                 * jnp.maximum(0.0, yy2 - yy1 + 1.0))
        iou = inter / jnp.maximum(areas + barea - inter, 1e-6)
        s = jnp.where(iou <= _THRESH, s, neg_inf)
        okf = ok.astype(jnp.float32)
        out_ref[0, t, 0] = bx1 * okf
        out_ref[0, t, 1] = by1 * okf
        out_ref[0, t, 2] = bx2 * okf
        out_ref[0, t, 3] = by2 * okf
        return s

    jax.lax.fori_loop(0, _POST_N, step, s0)


@functools.partial(jax.jit, static_argnums=())
def _nms_pallas(boxes, scores):
    B = boxes.shape[0]
    return pl.pallas_call(
        _nms_body,
        grid=(B,),
        in_specs=[
            pl.BlockSpec((1, 4, _ROWS, _COLS), lambda i: (i, 0, 0, 0)),
            pl.BlockSpec((1, _ROWS, _COLS), lambda i: (i, 0, 0)),
        ],
        out_specs=pl.BlockSpec((1, _POST_N, 4), lambda i: (i, 0, 0),
                               memory_space=pltpu.SMEM),
        out_shape=jax.ShapeDtypeStruct((B, _POST_N, 4), jnp.float32),
    )(boxes, scores)


def _bbox_transform_inv(boxes, deltas):
    widths = boxes[..., 2] - boxes[..., 0] + 1.0
    heights = boxes[..., 3] - boxes[..., 1] + 1.0
    ctr_x = boxes[..., 0] + 0.5 * widths
    ctr_y = boxes[..., 1] + 0.5 * heights
    dx, dy, dw, dh = (deltas[..., 0], deltas[..., 1],
                      deltas[..., 2], deltas[..., 3])
    pcx = dx * widths + ctr_x
    pcy = dy * heights + ctr_y
    pw = jnp.exp(dw) * widths
    ph = jnp.exp(dh) * heights
    return jnp.stack([pcx - 0.5 * pw, pcy - 0.5 * ph,
                      pcx + 0.5 * pw, pcy + 0.5 * ph], axis=-1)


def _clip_boxes(boxes, im_info):
    hmax = (im_info[:, 0] - 1.0)[:, None]
    wmax = (im_info[:, 1] - 1.0)[:, None]
    x1 = jnp.clip(boxes[..., 0], 0.0, wmax)
    y1 = jnp.clip(boxes[..., 1], 0.0, hmax)
    x2 = jnp.clip(boxes[..., 2], 0.0, wmax)
    y2 = jnp.clip(boxes[..., 3], 0.0, hmax)
    return jnp.stack([x1, y1, x2, y2], axis=-1)


def kernel(scores, bbox_deltas, im_info, cfg_key):
    del cfg_key
    anchors = jnp.asarray(_np_generate_anchors(), dtype=scores.dtype)
    sc = scores[:, _A:, :, :]
    B = bbox_deltas.shape[0]
    fh, fw = sc.shape[2], sc.shape[3]
    sx, sy = jnp.meshgrid(jnp.arange(fw) * _FEAT_STRIDE,
                          jnp.arange(fh) * _FEAT_STRIDE)
    shifts = jnp.stack([sx.ravel(), sy.ravel(), sx.ravel(), sy.ravel()],
                       axis=1).astype(scores.dtype)
    K = shifts.shape[0]
    all_anchors = (shifts[:, None, :] + anchors[None, :, :]).reshape(K * _A, 4)
    all_anchors = jnp.broadcast_to(all_anchors[None], (B, K * _A, 4))
    deltas = jnp.transpose(bbox_deltas, (0, 2, 3, 1)).reshape(B, -1, 4)
    scf = jnp.transpose(sc, (0, 2, 3, 1)).reshape(B, -1)
    proposals = _clip_boxes(_bbox_transform_inv(all_anchors, deltas), im_info)
    top_scores, order = jax.lax.top_k(scf, _PRE_N)
    props = jnp.take_along_axis(proposals, order[:, :, None], axis=1)

    pad_n = _NPAD - _PRE_N
    props_p = jnp.concatenate(
        [props, jnp.zeros((B, pad_n, 4), props.dtype)], axis=1)
    scores_p = jnp.concatenate(
        [top_scores, jnp.full((B, pad_n), -jnp.inf, top_scores.dtype)], axis=1)

    boxes_in = jnp.transpose(props_p, (0, 2, 1)).reshape(B, 4, _ROWS, _COLS)
    scores_in = scores_p.reshape(B, _ROWS, _COLS)

    kept = _nms_pallas(boxes_in, scores_in)
    bcol = jnp.broadcast_to(
        jnp.arange(B, dtype=kept.dtype)[:, None, None], (B, _POST_N, 1))
    return jnp.concatenate([bcol, kept], axis=2)


# speedup vs baseline: 1.6149x; 1.4146x over previous
import functools

import jax
import jax.numpy as jnp
import numpy as np
from jax.experimental import pallas as pl
from jax.experimental.pallas import tpu as pltpu

_A = 9
_FEAT_STRIDE = 16
_PRE_N = 6000
_POST_N = 300
_THRESH = 0.7
_NPAD = 6144
_B = 4
_ROWS = 8
_COLS = _NPAD // _ROWS


def _np_generate_anchors():
    base_size = 16
    ratios = np.array([0.5, 1.0, 2.0])
    scales = np.array([8.0, 16.0, 32.0])

    def whctrs(a):
        w = a[2] - a[0] + 1.0
        h = a[3] - a[1] + 1.0
        return w, h, a[0] + 0.5 * (w - 1.0), a[1] + 0.5 * (h - 1.0)

    def mkanchors(ws, hs, xc, yc):
        ws = ws[:, None]
        hs = hs[:, None]
        return np.hstack((xc - 0.5 * (ws - 1.0), yc - 0.5 * (hs - 1.0),
                          xc + 0.5 * (ws - 1.0), yc + 0.5 * (hs - 1.0)))

    base = np.array([1.0, 1.0, base_size, base_size], dtype=np.float64) - 1.0
    w, h, xc, yc = whctrs(base)
    size = w * h
    ws = np.round(np.sqrt(size / ratios))
    hs = np.round(ws * ratios)
    ra = mkanchors(ws, hs, xc, yc)
    outs = []
    for i in range(ra.shape[0]):
        w, h, xc, yc = whctrs(ra[i])
        outs.append(mkanchors(w * scales, h * scales, xc, yc))
    return np.vstack(outs)


def _nms_body(boxes_ref, scores_ref, out_ref, area_ref):
    ri = jax.lax.broadcasted_iota(jnp.int32, (_ROWS, _COLS), 0)
    ci = jax.lax.broadcasted_iota(jnp.int32, (_ROWS, _COLS), 1)
    fi = ri * _COLS + ci
    neg_inf = jnp.float32(-jnp.inf)
    zero = jnp.float32(0.0)
    big = jnp.int32(2**30)

    s0s = []
    for b in range(_B):
        x1 = boxes_ref[b, 0, :, :]
        y1 = boxes_ref[b, 1, :, :]
        x2 = boxes_ref[b, 2, :, :]
        y2 = boxes_ref[b, 3, :, :]
        area_ref[b, :, :] = (x2 - x1 + 1.0) * (y2 - y1 + 1.0)
        s0s.append(scores_ref[b, :, :])

    def step(t, ss):
        ms = [jnp.max(ss[b]) for b in range(_B)]
        eqs = [ss[b] == ms[b] for b in range(_B)]
        idxs = [jnp.min(jnp.where(eqs[b], fi, big)) for b in range(_B)]
        sels = [fi == idxs[b] for b in range(_B)]
        x1s = [boxes_ref[b, 0, :, :] for b in range(_B)]
        y1s = [boxes_ref[b, 1, :, :] for b in range(_B)]
        x2s = [boxes_ref[b, 2, :, :] for b in range(_B)]
        y2s = [boxes_ref[b, 3, :, :] for b in range(_B)]
        bx1s = [jnp.sum(jnp.where(sels[b], x1s[b], zero)) for b in range(_B)]
        by1s = [jnp.sum(jnp.where(sels[b], y1s[b], zero)) for b in range(_B)]
        bx2s = [jnp.sum(jnp.where(sels[b], x2s[b], zero)) for b in range(_B)]
        by2s = [jnp.sum(jnp.where(sels[b], y2s[b], zero)) for b in range(_B)]
        out = []
        for b in range(_B):
            x1, y1, x2, y2 = x1s[b], y1s[b], x2s[b], y2s[b]
            areas = area_ref[b, :, :]
            bx1, by1, bx2, by2 = bx1s[b], by1s[b], bx2s[b], by2s[b]
            barea = (bx2 - bx1 + 1.0) * (by2 - by1 + 1.0)
            xx1 = jnp.maximum(x1, bx1)
            yy1 = jnp.maximum(y1, by1)
            xx2 = jnp.minimum(x2, bx2)
            yy2 = jnp.minimum(y2, by2)
            inter = (jnp.maximum(0.0, xx2 - xx1 + 1.0)
                     * jnp.maximum(0.0, yy2 - yy1 + 1.0))
            iou = inter / jnp.maximum(areas + barea - inter, 1e-6)
            out.append(jnp.where(iou <= _THRESH, ss[b], neg_inf))
        for b in range(_B):
            okf = jnp.isfinite(ms[b]).astype(jnp.float32)
            out_ref[b, t, 0] = bx1s[b] * okf
            out_ref[b, t, 1] = by1s[b] * okf
            out_ref[b, t, 2] = bx2s[b] * okf
            out_ref[b, t, 3] = by2s[b] * okf
        return tuple(out)

    jax.lax.fori_loop(0, _POST_N, step, tuple(s0s))


@functools.partial(jax.jit, static_argnums=())
def _nms_pallas(boxes, scores):
    B = boxes.shape[0]
    return pl.pallas_call(
        _nms_body,
        out_specs=pl.BlockSpec(memory_space=pltpu.SMEM),
        out_shape=jax.ShapeDtypeStruct((B, _POST_N, 4), jnp.float32),
        scratch_shapes=[pltpu.VMEM((_B, _ROWS, _COLS), jnp.float32)],
    )(boxes, scores)


def _bbox_transform_inv(boxes, deltas):
    widths = boxes[..., 2] - boxes[..., 0] + 1.0
    heights = boxes[..., 3] - boxes[..., 1] + 1.0
    ctr_x = boxes[..., 0] + 0.5 * widths
    ctr_y = boxes[..., 1] + 0.5 * heights
    dx, dy, dw, dh = (deltas[..., 0], deltas[..., 1],
                      deltas[..., 2], deltas[..., 3])
    pcx = dx * widths + ctr_x
    pcy = dy * heights + ctr_y
    pw = jnp.exp(dw) * widths
    ph = jnp.exp(dh) * heights
    return jnp.stack([pcx - 0.5 * pw, pcy - 0.5 * ph,
                      pcx + 0.5 * pw, pcy + 0.5 * ph], axis=-1)


def _clip_boxes(boxes, im_info):
    hmax = (im_info[:, 0] - 1.0)[:, None]
    wmax = (im_info[:, 1] - 1.0)[:, None]
    x1 = jnp.clip(boxes[..., 0], 0.0, wmax)
    y1 = jnp.clip(boxes[..., 1], 0.0, hmax)
    x2 = jnp.clip(boxes[..., 2], 0.0, wmax)
    y2 = jnp.clip(boxes[..., 3], 0.0, hmax)
    return jnp.stack([x1, y1, x2, y2], axis=-1)


def kernel(scores, bbox_deltas, im_info, cfg_key):
    del cfg_key
    anchors = jnp.asarray(_np_generate_anchors(), dtype=scores.dtype)
    sc = scores[:, _A:, :, :]
    B = bbox_deltas.shape[0]
    fh, fw = sc.shape[2], sc.shape[3]
    sx, sy = jnp.meshgrid(jnp.arange(fw) * _FEAT_STRIDE,
                          jnp.arange(fh) * _FEAT_STRIDE)
    shifts = jnp.stack([sx.ravel(), sy.ravel(), sx.ravel(), sy.ravel()],
                       axis=1).astype(scores.dtype)
    K = shifts.shape[0]
    all_anchors = (shifts[:, None, :] + anchors[None, :, :]).reshape(K * _A, 4)
    all_anchors = jnp.broadcast_to(all_anchors[None], (B, K * _A, 4))
    deltas = jnp.transpose(bbox_deltas, (0, 2, 3, 1)).reshape(B, -1, 4)
    scf = jnp.transpose(sc, (0, 2, 3, 1)).reshape(B, -1)
    proposals = _clip_boxes(_bbox_transform_inv(all_anchors, deltas), im_info)
    top_scores, order = jax.lax.top_k(scf, _PRE_N)
    props = jnp.take_along_axis(proposals, order[:, :, None], axis=1)

    pad_n = _NPAD - _PRE_N
    props_p = jnp.concatenate(
        [props, jnp.zeros((B, pad_n, 4), props.dtype)], axis=1)
    scores_p = jnp.concatenate(
        [top_scores, jnp.full((B, pad_n), -jnp.inf, top_scores.dtype)], axis=1)

    boxes_in = jnp.transpose(props_p, (0, 2, 1)).reshape(B, 4, _ROWS, _COLS)
    scores_in = scores_p.reshape(B, _ROWS, _COLS)

    kept = _nms_pallas(boxes_in, scores_in)
    bcol = jnp.broadcast_to(
        jnp.arange(B, dtype=kept.dtype)[:, None, None], (B, _POST_N, 1))
    return jnp.concatenate([bcol, kept], axis=2)
